# baseline (speedup 1.0000x reference)
import functools

import jax
import jax.numpy as jnp
from jax import lax
from jax.experimental import pallas as pl
from jax.experimental.pallas import tpu as pltpu
from jax.experimental.pallas import tpu_sc as plsc

VOCAB = 256
POS = 512
D = 768
B = 1024
L = 200

NC = 2
NS = 16
NW = NC * NS

TOTAL = B * L
B_SC = 512
B_TC = B - B_SC
SC_TOTAL = B_SC * L
PER_W = SC_TOTAL // NW
K = 40
NCHUNK = PER_W // K
NBUF = 4


LBLK = 8
NLBLK = L // LBLK
PADV = 512


def _table_body(p_blk_ref, w_ref, b_ref, e_ref, t_ref, stk_ref, pe_ref):
    i = pl.program_id(0)
    contract = (((1,), (1,)), ((), ()))

    @pl.when(i == 0)
    def _():
        pe_ref[...] = lax.dot_general(
            e_ref[...], w_ref[...], contract,
            preferred_element_type=jnp.float32)
        stk_ref[0:VOCAB, :] = pe_ref[...].astype(jnp.bfloat16)
        stk_ref[VOCAB + L:PADV, :] = jnp.zeros(
            (PADV - VOCAB - L, D), jnp.bfloat16)

    pp = lax.dot_general(
        p_blk_ref[...], w_ref[...], contract,
        preferred_element_type=jnp.float32) + b_ref[...]
    t_ref[...] = pe_ref[...][None, :, :] + pp[:, None, :]
    stk_ref[pl.ds(VOCAB + i * LBLK, LBLK), :] = pp.astype(jnp.bfloat16)


def _build_table(pos_table, W, b2, embed_table):
    return pl.pallas_call(
        _table_body,
        grid=(NLBLK,),
        in_specs=[
            pl.BlockSpec((LBLK, D), lambda i: (i, 0)),
            pl.BlockSpec((D, D), lambda i: (0, 0)),
            pl.BlockSpec((1, D), lambda i: (0, 0)),
            pl.BlockSpec((VOCAB, D), lambda i: (0, 0)),
        ],
        out_specs=[
            pl.BlockSpec((LBLK, VOCAB, D), lambda i: (i, 0, 0)),
            pl.BlockSpec((PADV, D), lambda i: (0, 0)),
        ],
        out_shape=[
            jax.ShapeDtypeStruct((L, VOCAB, D), jnp.float32),
            jax.ShapeDtypeStruct((PADV, D), jnp.bfloat16),
        ],
        scratch_shapes=[pltpu.VMEM((VOCAB, D), jnp.float32)],
    )(pos_table, W, b2, embed_table)


TBLK = 16


def _onehot_body(ids_ref, stk_ref, alias_ref, out_ref):
    del alias_ref
    ids3 = ids_ref[...]
    col = lax.broadcasted_iota(jnp.int32, (TBLK, L, PADV), 2)
    pos = lax.broadcasted_iota(jnp.int32, (TBLK, L, PADV), 1)
    sel = (col == ids3[:, :, None]) | (col == pos + VOCAB)
    oh = sel.astype(jnp.bfloat16).reshape(TBLK * L, PADV)
    out_ref[...] = lax.dot_general(
        oh, stk_ref[...], (((1,), (0,)), ((), ())),
        preferred_element_type=jnp.float32)


def _onehot_fill(ids, stk, out_sc):
    return pl.pallas_call(
        _onehot_body,
        grid=(B_TC // TBLK,),
        in_specs=[
            pl.BlockSpec((TBLK, L), lambda i: (B_SC // TBLK + i, 0)),
            pl.BlockSpec((PADV, D), lambda i: (0, 0)),
            pl.BlockSpec(memory_space=pltpu.MemorySpace.HBM),
        ],
        out_specs=pl.BlockSpec((TBLK * L, D), lambda i: (B_SC // TBLK + i, 0)),
        out_shape=jax.ShapeDtypeStruct((TOTAL, D), jnp.float32),
        input_output_aliases={2: 0},
    )(ids, stk, out_sc)


def _gather_body(table_hbm, idx_hbm, out_hbm, idx_all, *bufs):
    rows = bufs[:NBUF]
    gsem = bufs[NBUF:2 * NBUF]
    ssem = bufs[2 * NBUF:3 * NBUF]
    wid = lax.axis_index("s") * NC + lax.axis_index("c")
    base = wid * PER_W

    pltpu.sync_copy(idx_hbm.at[pl.ds(base, PER_W)], idx_all)

    def g_issue(i, b):
        pltpu.async_copy(
            table_hbm.at[idx_all.at[pl.ds(i * K, K)]], rows[b], gsem[b])

    def g_wait(i, b):
        pltpu.make_async_copy(
            table_hbm.at[idx_all.at[pl.ds(i * K, K)]], rows[b],
            gsem[b]).wait()

    def s_issue(i, b):
        pltpu.async_copy(rows[b], out_hbm.at[pl.ds(base + i * K, K)],
                         ssem[b])

    def s_wait(i, b):
        pltpu.make_async_copy(rows[b], out_hbm.at[pl.ds(base + i * K, K)],
                              ssem[b]).wait()

    g_issue(0, 0)
    g_issue(1, 1)

    def quad(j, _):
        for r in range(NBUF):
            i = NBUF * j + r

            @pl.when(i + 2 < NCHUNK)
            def _():
                @pl.when(i >= 2)
                def _():
                    s_wait(i - 2, (r + 2) % NBUF)
                g_issue(i + 2, (r + 2) % NBUF)

            g_wait(i, r)
            s_issue(i, r)
        return 0

    lax.fori_loop(0, NCHUNK // NBUF, quad, 0)
    for t in range(NBUF):
        i = NCHUNK - NBUF + t
        s_wait(i, i % NBUF)


@functools.cache
def _gather_rows():
    return pl.kernel(
        _gather_body,
        out_type=jax.ShapeDtypeStruct((TOTAL, D), jnp.float32),
        mesh=plsc.VectorSubcoreMesh(
            core_axis_name="c", subcore_axis_name="s",
            num_cores=NC, num_subcores=NS),
        scratch_types=[
            pltpu.VMEM((PER_W,), jnp.int32),
            *[pltpu.VMEM((K, D), jnp.float32) for _ in range(NBUF)],
            *[pltpu.SemaphoreType.DMA for _ in range(2 * NBUF)],
        ],
    )


@jax.jit
def kernel(char_ids, embed_table, pos_table, W, b):
    table, stk = _build_table(
        pos_table[:L], W, b.reshape(1, D), embed_table)
    table = table.reshape(L * VOCAB, D)
    ids = char_ids.astype(jnp.int32)
    flat_idx = (
        ids[:B_SC] + (jnp.arange(L, dtype=jnp.int32) * VOCAB)[None, :]
    ).reshape(-1)
    out_sc = _gather_rows()(table, flat_idx)
    out = _onehot_fill(ids, stk, out_sc)
    return out.reshape(B, L, D)


# speedup vs baseline: 1.0306x; 1.0306x over previous
import functools

import jax
import jax.numpy as jnp
from jax import lax
from jax.experimental import pallas as pl
from jax.experimental.pallas import tpu as pltpu
from jax.experimental.pallas import tpu_sc as plsc

VOCAB = 256
POS = 512
D = 768
B = 1024
L = 200

NC = 2
NS = 16
NW = NC * NS

TOTAL = B * L
B_SC = 512
B_TC = B - B_SC
SC_TOTAL = B_SC * L
PER_W = SC_TOTAL // NW
K = 40
NCHUNK = PER_W // K
NBUF = 4


LBLK = 8
NLBLK = L // LBLK
PADV = 512


def _table_body(p_blk_ref, w_ref, b_ref, e_ref, t_ref, peb_ref, pp_ref,
                pe_ref):
    i = pl.program_id(0)
    contract = (((1,), (1,)), ((), ()))

    @pl.when(i == 0)
    def _():
        pe_ref[...] = lax.dot_general(
            e_ref[...], w_ref[...], contract,
            preferred_element_type=jnp.float32)
        peb_ref[...] = pe_ref[...].astype(jnp.bfloat16)

    pp = lax.dot_general(
        p_blk_ref[...], w_ref[...], contract,
        preferred_element_type=jnp.float32) + b_ref[...]
    t_ref[...] = pe_ref[...][None, :, :] + pp[:, None, :]
    pp_ref[...] = pp


def _build_table(pos_table, W, b2, embed_table):
    return pl.pallas_call(
        _table_body,
        grid=(NLBLK,),
        in_specs=[
            pl.BlockSpec((LBLK, D), lambda i: (i, 0)),
            pl.BlockSpec((D, D), lambda i: (0, 0)),
            pl.BlockSpec((1, D), lambda i: (0, 0)),
            pl.BlockSpec((VOCAB, D), lambda i: (0, 0)),
        ],
        out_specs=[
            pl.BlockSpec((LBLK, VOCAB, D), lambda i: (i, 0, 0)),
            pl.BlockSpec((VOCAB, D), lambda i: (0, 0)),
            pl.BlockSpec((LBLK, D), lambda i: (i, 0)),
        ],
        out_shape=[
            jax.ShapeDtypeStruct((L, VOCAB, D), jnp.float32),
            jax.ShapeDtypeStruct((VOCAB, D), jnp.bfloat16),
            jax.ShapeDtypeStruct((L, D), jnp.float32),
        ],
        scratch_shapes=[pltpu.VMEM((VOCAB, D), jnp.float32)],
    )(pos_table, W, b2, embed_table)


TBLK = 16


def _onehot_body(ids_ref, pe_ref, pp_ref, alias_ref, out_ref):
    del alias_ref
    ids3 = ids_ref[...]
    col = lax.broadcasted_iota(jnp.int32, (TBLK, L, VOCAB), 2)
    oh = (col == ids3[:, :, None]).astype(jnp.bfloat16).reshape(
        TBLK * L, VOCAB)
    acc = lax.dot_general(
        oh, pe_ref[...], (((1,), (0,)), ((), ())),
        preferred_element_type=jnp.float32)
    acc = acc.reshape(TBLK, L, D) + pp_ref[...][None]
    out_ref[...] = acc.reshape(TBLK * L, D)


def _onehot_fill(ids, pe_b, pp, out_sc):
    return pl.pallas_call(
        _onehot_body,
        grid=(B_TC // TBLK,),
        in_specs=[
            pl.BlockSpec((TBLK, L), lambda i: (B_SC // TBLK + i, 0)),
            pl.BlockSpec((VOCAB, D), lambda i: (0, 0)),
            pl.BlockSpec((L, D), lambda i: (0, 0)),
            pl.BlockSpec(memory_space=pltpu.MemorySpace.HBM),
        ],
        out_specs=pl.BlockSpec((TBLK * L, D), lambda i: (B_SC // TBLK + i, 0)),
        out_shape=jax.ShapeDtypeStruct((TOTAL, D), jnp.float32),
        input_output_aliases={3: 0},
    )(ids, pe_b, pp, out_sc)


def _gather_body(table_hbm, idx_hbm, out_hbm, idx_all, *bufs):
    rows = bufs[:NBUF]
    gsem = bufs[NBUF:2 * NBUF]
    ssem = bufs[2 * NBUF:3 * NBUF]
    wid = lax.axis_index("s") * NC + lax.axis_index("c")
    base = wid * PER_W

    pltpu.sync_copy(idx_hbm.at[pl.ds(base, PER_W)], idx_all)

    def g_issue(i, b):
        pltpu.async_copy(
            table_hbm.at[idx_all.at[pl.ds(i * K, K)]], rows[b], gsem[b])

    def g_wait(i, b):
        pltpu.make_async_copy(
            table_hbm.at[idx_all.at[pl.ds(i * K, K)]], rows[b],
            gsem[b]).wait()

    def s_issue(i, b):
        pltpu.async_copy(rows[b], out_hbm.at[pl.ds(base + i * K, K)],
                         ssem[b])

    def s_wait(i, b):
        pltpu.make_async_copy(rows[b], out_hbm.at[pl.ds(base + i * K, K)],
                              ssem[b]).wait()

    g_issue(0, 0)
    g_issue(1, 1)

    def quad(j, _):
        for r in range(NBUF):
            i = NBUF * j + r

            @pl.when(i + 2 < NCHUNK)
            def _():
                @pl.when(i >= 2)
                def _():
                    s_wait(i - 2, (r + 2) % NBUF)
                g_issue(i + 2, (r + 2) % NBUF)

            g_wait(i, r)
            s_issue(i, r)
        return 0

    lax.fori_loop(0, NCHUNK // NBUF, quad, 0)
    for t in range(NBUF):
        i = NCHUNK - NBUF + t
        s_wait(i, i % NBUF)


@functools.cache
def _gather_rows():
    return pl.kernel(
        _gather_body,
        out_type=jax.ShapeDtypeStruct((TOTAL, D), jnp.float32),
        mesh=plsc.VectorSubcoreMesh(
            core_axis_name="c", subcore_axis_name="s",
            num_cores=NC, num_subcores=NS),
        scratch_types=[
            pltpu.VMEM((PER_W,), jnp.int32),
            *[pltpu.VMEM((K, D), jnp.float32) for _ in range(NBUF)],
            *[pltpu.SemaphoreType.DMA for _ in range(2 * NBUF)],
        ],
    )


@jax.jit
def kernel(char_ids, embed_table, pos_table, W, b):
    table, pe_b, pp = _build_table(
        pos_table[:L], W, b.reshape(1, D), embed_table)
    table = table.reshape(L * VOCAB, D)
    ids = char_ids.astype(jnp.int32)
    flat_idx = (
        ids[:B_SC] + (jnp.arange(L, dtype=jnp.int32) * VOCAB)[None, :]
    ).reshape(-1)
    out_sc = _gather_rows()(table, flat_idx)
    out = _onehot_fill(ids, pe_b, pp, out_sc)
    return out.reshape(B, L, D)


# speedup vs baseline: 1.1187x; 1.0854x over previous
import functools

import jax
import jax.numpy as jnp
from jax import lax
from jax.experimental import pallas as pl
from jax.experimental.pallas import tpu as pltpu
from jax.experimental.pallas import tpu_sc as plsc

VOCAB = 256
POS = 512
D = 768
B = 1024
L = 200

NC = 2
NS = 16
NW = NC * NS

TOTAL = B * L
B_SC = 384
B_TC = B - B_SC
SC_TOTAL = B_SC * L
PER_W = SC_TOTAL // NW
K = 40
NCHUNK = PER_W // K
NBUF = 4


LBLK = 8
NLBLK = L // LBLK
PADV = 512


def _table_body(p_blk_ref, w_ref, b_ref, e_ref, t_ref, peb_ref, pp_ref,
                pe_ref):
    i = pl.program_id(0)
    contract = (((1,), (1,)), ((), ()))

    @pl.when(i == 0)
    def _():
        pe_ref[...] = lax.dot_general(
            e_ref[...], w_ref[...], contract,
            preferred_element_type=jnp.float32)
        peb_ref[...] = pe_ref[...].astype(jnp.bfloat16)

    pp = lax.dot_general(
        p_blk_ref[...], w_ref[...], contract,
        preferred_element_type=jnp.float32) + b_ref[...]
    t_ref[...] = pe_ref[...][None, :, :] + pp[:, None, :]
    pp_ref[...] = pp


def _build_table(pos_table, W, b2, embed_table):
    return pl.pallas_call(
        _table_body,
        grid=(NLBLK,),
        in_specs=[
            pl.BlockSpec((LBLK, D), lambda i: (i, 0)),
            pl.BlockSpec((D, D), lambda i: (0, 0)),
            pl.BlockSpec((1, D), lambda i: (0, 0)),
            pl.BlockSpec((VOCAB, D), lambda i: (0, 0)),
        ],
        out_specs=[
            pl.BlockSpec((LBLK, VOCAB, D), lambda i: (i, 0, 0)),
            pl.BlockSpec((VOCAB, D), lambda i: (0, 0)),
            pl.BlockSpec((LBLK, D), lambda i: (i, 0)),
        ],
        out_shape=[
            jax.ShapeDtypeStruct((L, VOCAB, D), jnp.float32),
            jax.ShapeDtypeStruct((VOCAB, D), jnp.bfloat16),
            jax.ShapeDtypeStruct((L, D), jnp.float32),
        ],
        scratch_shapes=[pltpu.VMEM((VOCAB, D), jnp.float32)],
    )(pos_table, W, b2, embed_table)


TBLK = 16


def _onehot_body(ids_ref, pe_ref, pp_ref, alias_ref, out_ref):
    del alias_ref
    ids3 = ids_ref[...]
    col = lax.broadcasted_iota(jnp.int32, (TBLK, L, VOCAB), 2)
    oh = (col == ids3[:, :, None]).astype(jnp.bfloat16).reshape(
        TBLK * L, VOCAB)
    acc = lax.dot_general(
        oh, pe_ref[...], (((1,), (0,)), ((), ())),
        preferred_element_type=jnp.float32)
    acc = acc.reshape(TBLK, L, D) + pp_ref[...][None]
    out_ref[...] = acc.reshape(TBLK * L, D)


def _onehot_fill(ids, pe_b, pp, out_sc):
    return pl.pallas_call(
        _onehot_body,
        grid=(B_TC // TBLK,),
        in_specs=[
            pl.BlockSpec((TBLK, L), lambda i: (B_SC // TBLK + i, 0)),
            pl.BlockSpec((VOCAB, D), lambda i: (0, 0)),
            pl.BlockSpec((L, D), lambda i: (0, 0)),
            pl.BlockSpec(memory_space=pltpu.MemorySpace.HBM),
        ],
        out_specs=pl.BlockSpec((TBLK * L, D), lambda i: (B_SC // TBLK + i, 0)),
        out_shape=jax.ShapeDtypeStruct((TOTAL, D), jnp.float32),
        input_output_aliases={3: 0},
    )(ids, pe_b, pp, out_sc)


def _gather_body(table_hbm, idx_hbm, out_hbm, idx_all, *bufs):
    rows = bufs[:NBUF]
    gsem = bufs[NBUF:2 * NBUF]
    ssem = bufs[2 * NBUF:3 * NBUF]
    wid = lax.axis_index("s") * NC + lax.axis_index("c")
    base = wid * PER_W

    pltpu.sync_copy(idx_hbm.at[pl.ds(base, PER_W)], idx_all)

    def g_issue(i, b):
        pltpu.async_copy(
            table_hbm.at[idx_all.at[pl.ds(i * K, K)]], rows[b], gsem[b])

    def g_wait(i, b):
        pltpu.make_async_copy(
            table_hbm.at[idx_all.at[pl.ds(i * K, K)]], rows[b],
            gsem[b]).wait()

    def s_issue(i, b):
        pltpu.async_copy(rows[b], out_hbm.at[pl.ds(base + i * K, K)],
                         ssem[b])

    def s_wait(i, b):
        pltpu.make_async_copy(rows[b], out_hbm.at[pl.ds(base + i * K, K)],
                              ssem[b]).wait()

    g_issue(0, 0)
    g_issue(1, 1)

    def quad(j, _):
        for r in range(NBUF):
            i = NBUF * j + r

            @pl.when(i + 2 < NCHUNK)
            def _():
                @pl.when(i >= 2)
                def _():
                    s_wait(i - 2, (r + 2) % NBUF)
                g_issue(i + 2, (r + 2) % NBUF)

            g_wait(i, r)
            s_issue(i, r)
        return 0

    lax.fori_loop(0, NCHUNK // NBUF, quad, 0)
    for t in range(NBUF):
        i = NCHUNK - NBUF + t
        s_wait(i, i % NBUF)


@functools.cache
def _gather_rows():
    return pl.kernel(
        _gather_body,
        out_type=jax.ShapeDtypeStruct((TOTAL, D), jnp.float32),
        mesh=plsc.VectorSubcoreMesh(
            core_axis_name="c", subcore_axis_name="s",
            num_cores=NC, num_subcores=NS),
        scratch_types=[
            pltpu.VMEM((PER_W,), jnp.int32),
            *[pltpu.VMEM((K, D), jnp.float32) for _ in range(NBUF)],
            *[pltpu.SemaphoreType.DMA for _ in range(2 * NBUF)],
        ],
    )


@jax.jit
def kernel(char_ids, embed_table, pos_table, W, b):
    table, pe_b, pp = _build_table(
        pos_table[:L], W, b.reshape(1, D), embed_table)
    table = table.reshape(L * VOCAB, D)
    ids = char_ids.astype(jnp.int32)
    flat_idx = (
        ids[:B_SC] + (jnp.arange(L, dtype=jnp.int32) * VOCAB)[None, :]
    ).reshape(-1)
    out_sc = _gather_rows()(table, flat_idx)
    out = _onehot_fill(ids, pe_b, pp, out_sc)
    return out.reshape(B, L, D)
